# baseline (speedup 1.0000x reference)
import jax
import jax.numpy as jnp
from jax import lax
from jax.experimental import pallas as pl
from jax.experimental.pallas import tpu as pltpu
from jax.experimental.pallas import tpu_sc as plsc

_N = 10000
_E = 320000
_HID = 128
_OUT = 64

_EH = _E // 2
_ETS = _E // 16
_BS = 400
_BF = 1280
_NB = 125

_mesh = plsc.VectorSubcoreMesh(
    core_axis_name="c", subcore_axis_name="s", num_cores=2, num_subcores=16)

_sc_params = pltpu.CompilerParams(
    needs_layout_passes=False, use_tc_tiling_on_sc=False)


def _sum_body(table, src, dst, zinit, out_s, out_d, sidx0, dvec0, msg0,
              sidx1, dvec1, msg1, deg, shared, sem0, sem1):
    c = lax.axis_index("c")
    s = lax.axis_index("s")

    @pl.when(s == 0)
    def _():
        pltpu.sync_copy(zinit, shared)

    zero16 = jnp.zeros((16,), jnp.float32)

    def zdeg(i, _):
        deg[pl.ds(i * 16, 16)] = zero16
        return 0

    lax.fori_loop(0, _N // 16, zdeg, 0)
    plsc.subcore_barrier()

    ebase = s * _ETS
    ones16 = jnp.ones((16,), jnp.float32)
    chunks = ((0, 128), (128, 128), (256, 128), (384, 16))
    bufs = ((sidx0, dvec0, msg0, sem0), (sidx1, dvec1, msg1, sem1))

    def stage(b, sidx, dvec, msg, sem):
        base = ebase + b * _BS
        pltpu.sync_copy(src.at[pl.ds(base, _BS)], sidx)
        pltpu.sync_copy(dst.at[pl.ds(base, _BS)], dvec)

        def adj(i, _):
            for j in range(5):
                v = sidx[pl.ds((i * 5 + j) * 16, 16)]
                sidx[pl.ds((i * 5 + j) * 16, 16)] = v * 2 + c
            return 0

        lax.fori_loop(0, 5, adj, 0)
        for k, sz in chunks:
            pltpu.async_copy(table.at[sidx.at[pl.ds(k, sz)]],
                             msg.at[pl.ds(k, sz)], sem)

    def process(sidx, dvec, msg, sem):
        for k, sz in chunks:
            pltpu.make_async_copy(table.at[sidx.at[pl.ds(k, sz)]],
                                  msg.at[pl.ds(k, sz)], sem).wait()
        for k, sz in chunks:
            pltpu.sync_copy(msg.at[pl.ds(k, sz)],
                            shared.at[dvec.at[pl.ds(k, sz)]], add=True)

        @pl.when(c == 0)
        def _():
            def grpd(g, _):
                d = dvec[pl.ds(g * 16, 16)]
                plsc.addupdate_scatter(deg, [d], ones16)
                return 0

            lax.fori_loop(0, _BS // 16, grpd, 0)

    npair = _ETS // _BS // 2
    stage(0, *bufs[0])

    def pair(i, _):
        stage(i * 2 + 1, *bufs[1])
        process(*bufs[0])

        @pl.when(i < npair - 1)
        def _():
            stage(i * 2 + 2, *bufs[0])

        process(*bufs[1])
        return 0

    lax.fori_loop(0, npair, pair, 0)
    plsc.subcore_barrier()

    @pl.when(s == 0)
    def _():
        pltpu.sync_copy(shared, out_s.at[pl.ds(0, _N), pl.ds(c * 64, 64)])

    @pl.when(c == 0)
    def _():
        pltpu.sync_copy(deg, out_d.at[s])


def _min_body(table, src, dst, iinit, out_m, acc, tmp,
              sidx0, dvec0, msg0, sidx1, dvec1, msg1, sidx2, dvec2, msg2,
              semi0, semi1, semi2, semg0, semg1, semg2):
    c = lax.axis_index("c")
    s = lax.axis_index("s")
    fc = s
    eh = c
    pltpu.sync_copy(iinit, acc)
    iota = lax.iota(jnp.int32, 16)
    ebase = eh * _EH
    bufs = ((sidx0, dvec0, msg0, semi0, semg0),
            (sidx1, dvec1, msg1, semi1, semg1),
            (sidx2, dvec2, msg2, semi2, semg2))

    def fire_idx(b, slot):
        sidx, dvec, _, semi, _ = bufs[slot]
        base = ebase + b * _BF
        pltpu.async_copy(src.at[pl.ds(base, _BF)], sidx, semi)
        pltpu.async_copy(dst.at[pl.ds(base, _BF)], dvec, semi)

    def phase_a(b, slot):
        sidx, dvec, msg, semi, semg = bufs[slot]
        base = ebase + b * _BF
        pltpu.make_async_copy(src.at[pl.ds(base, _BF)], sidx, semi).wait()
        pltpu.make_async_copy(dst.at[pl.ds(base, _BF)], dvec, semi).wait()

        def adj(i, _):
            for j in range(4):
                v = sidx[pl.ds((i * 4 + j) * 16, 16)]
                sidx[pl.ds((i * 4 + j) * 16, 16)] = v * 16 + fc
            return 0

        lax.fori_loop(0, _BF // 64, adj, 0)
        for k in range(_BF // 128):
            pltpu.async_copy(
                table.at[sidx.at[pl.ds(k * 128, 128)]],
                msg.at[pl.ds(k * 128, 128)], semg)

    def accum(dvec, msg):
        npg = _BF // 32
        iota16 = iota + 16

        def checkpair(pi):
            dA = dvec[pl.ds(pi * 32, 16)]
            dB = dvec[pl.ds(pi * 32 + 16, 16)]
            plsc.store_scatter(tmp, [dA], iota)
            plsc.store_scatter(tmp, [dB], iota16)
            rbA = plsc.load_gather(tmp, [dA])
            rbB = plsc.load_gather(tmp, [dB])
            return dA, dB, jnp.any((rbA != iota) | (rbB != iota16))

        def grp(pi, carry):
            dA, dB, dup = carry
            nxt = checkpair(jnp.minimum(pi + 1, npg - 1))
            eA = pi * 32 + iota
            eB = eA + 16

            @pl.when(jnp.logical_not(dup))
            def _():
                @plsc.parallel_loop(0, 8, step=1, unroll=4)
                def _(f):
                    fv = jnp.full((16,), f, jnp.int32)
                    colA = plsc.load_gather(msg, [eA, fv])
                    colB = plsc.load_gather(msg, [eB, fv])
                    curA = plsc.load_gather(acc, [dA, fv])
                    curB = plsc.load_gather(acc, [dB, fv])
                    plsc.store_scatter(acc, [dA, fv], jnp.minimum(curA, colA))
                    plsc.store_scatter(acc, [dB, fv], jnp.minimum(curB, colB))

            @pl.when(dup)
            def _():
                for d, e in ((dA, eA), (dB, eB)):
                    for f in range(8):
                        fv = jnp.full((16,), f, jnp.int32)
                        col = plsc.load_gather(msg, [e, fv])

                        def body(a):
                            plsc.store_scatter(acc, [d, fv], col, mask=a)
                            rb2 = plsc.load_gather(acc, [d, fv])
                            return a & (col < rb2)

                        a0 = col < plsc.load_gather(acc, [d, fv])
                        lax.while_loop(lambda a: jnp.any(a), body, a0)

            return nxt

        lax.fori_loop(0, npg, grp, checkpair(0))

    def phase_b(b, slot):
        sidx, dvec, msg, _, semg = bufs[slot]
        base = ebase + b * _BF
        del base
        for k in range(_BF // 128):
            pltpu.make_async_copy(
                table.at[sidx.at[pl.ds(k * 128, 128)]],
                msg.at[pl.ds(k * 128, 128)], semg).wait()
        accum(dvec, msg)

    fire_idx(0, 0)
    fire_idx(1, 1)
    phase_a(0, 0)

    def triple(i, _):
        for j in range(3):
            b = i * 3 + j
            fire_idx(b + 2, (j + 2) % 3)
            phase_a(b + 1, (j + 1) % 3)
            phase_b(b, j)
        return 0

    lax.fori_loop(0, (_NB - 2) // 3, triple, 0)
    phase_a(_NB - 1, (_NB - 1) % 3)
    phase_b(_NB - 2, (_NB - 2) % 3)
    phase_b(_NB - 1, (_NB - 1) % 3)

    pltpu.sync_copy(acc, out_m.at[eh, pl.ds(0, _N), pl.ds(fc * 8, 8)])


_seg_sum = pl.kernel(
    _sum_body,
    out_type=(jax.ShapeDtypeStruct((_N, _HID), jnp.float32),
              jax.ShapeDtypeStruct((16, _N), jnp.float32)),
    mesh=_mesh,
    compiler_params=_sc_params,
    scratch_types=[
        pltpu.VMEM((_BS,), jnp.int32),
        pltpu.VMEM((_BS,), jnp.int32),
        pltpu.VMEM((_BS, 64), jnp.float32),
        pltpu.VMEM((_BS,), jnp.int32),
        pltpu.VMEM((_BS,), jnp.int32),
        pltpu.VMEM((_BS, 64), jnp.float32),
        pltpu.VMEM((_N,), jnp.float32),
        pltpu.VMEM_SHARED((_N, 64), jnp.float32),
        pltpu.SemaphoreType.DMA,
        pltpu.SemaphoreType.DMA,
    ],
)

_seg_min = pl.kernel(
    _min_body,
    out_type=jax.ShapeDtypeStruct((2, _N, _HID), jnp.float32),
    mesh=_mesh,
    compiler_params=_sc_params,
    scratch_types=[
        pltpu.VMEM((_N, 8), jnp.float32),
        pltpu.VMEM((_N,), jnp.int32),
        pltpu.VMEM((_BF,), jnp.int32),
        pltpu.VMEM((_BF,), jnp.int32),
        pltpu.VMEM((_BF, 8), jnp.float32),
        pltpu.VMEM((_BF,), jnp.int32),
        pltpu.VMEM((_BF,), jnp.int32),
        pltpu.VMEM((_BF, 8), jnp.float32),
        pltpu.VMEM((_BF,), jnp.int32),
        pltpu.VMEM((_BF,), jnp.int32),
        pltpu.VMEM((_BF, 8), jnp.float32),
        pltpu.SemaphoreType.DMA,
        pltpu.SemaphoreType.DMA,
        pltpu.SemaphoreType.DMA,
        pltpu.SemaphoreType.DMA,
        pltpu.SemaphoreType.DMA,
        pltpu.SemaphoreType.DMA,
    ],
)


def _mm_tanh_body(x_ref, w_ref, b_ref, o_ref):
    o_ref[...] = jnp.tanh(
        jnp.dot(x_ref[...], w_ref[...], preferred_element_type=jnp.float32)
        + b_ref[...])


def _mid_body(sp_ref, dg_ref, w_ref, b_ref, o_ref):
    s = sp_ref[...]
    deg = jnp.sum(dg_ref[...], axis=0)
    m = s / jnp.maximum(deg, 1.0)[:, None]
    o_ref[...] = jnp.tanh(
        jnp.dot(m, w_ref[...], preferred_element_type=jnp.float32)
        + b_ref[...])


def _cls_body(ap_ref, w1_ref, b1_ref, w2_ref, b2_ref, o_ref):
    agg = jnp.minimum(ap_ref[0], ap_ref[1])
    c1 = jnp.tanh(
        jnp.dot(agg, w1_ref[...], preferred_element_type=jnp.float32)
        + b1_ref[...])
    o_ref[...] = (
        jnp.dot(c1, w2_ref[...], preferred_element_type=jnp.float32)
        + b2_ref[...])


_mm_tanh = pl.pallas_call(
    _mm_tanh_body,
    out_shape=jax.ShapeDtypeStruct((_N, _HID), jnp.float32),
)

_mid = pl.pallas_call(
    _mid_body,
    out_shape=jax.ShapeDtypeStruct((_N, _HID), jnp.float32),
)

_cls = pl.pallas_call(
    _cls_body,
    out_shape=jax.ShapeDtypeStruct((_N, _OUT), jnp.float32),
)


def kernel(x, x_struct, x_e, edge_index, W1, b1, W2, b2, Wc1, bc1, Wc2, bc2):
    del x_struct, x_e
    src = edge_index[0]
    dst = edge_index[1]

    h1 = _mm_tanh(x, W1, b1.reshape(1, _HID))
    zinit = jnp.zeros((_N, 64), jnp.float32)
    s_part, d_part = _seg_sum(h1.reshape(_N * 2, 64), src, dst, zinit)
    h2 = _mid(s_part, d_part, W2, b2.reshape(1, _HID))
    iinit = jnp.full((_N, 8), jnp.inf, jnp.float32)
    a_part = _seg_min(h2.reshape(_N * 16, 8), src, dst, iinit)
    out = _cls(a_part, Wc1, bc1.reshape(1, _HID), Wc2, bc2.reshape(1, _OUT))
    return out


# speedup vs baseline: 1.2769x; 1.2769x over previous
import jax
import jax.numpy as jnp
from jax import lax
from jax.experimental import pallas as pl
from jax.experimental.pallas import tpu as pltpu
from jax.experimental.pallas import tpu_sc as plsc

_N = 10000
_E = 320000
_HID = 128
_OUT = 64

_EH = _E // 2
_ETS = _E // 16
_BS = 400
_BF = 1280
_NB = 125

_mesh = plsc.VectorSubcoreMesh(
    core_axis_name="c", subcore_axis_name="s", num_cores=2, num_subcores=16)

_sc_params = pltpu.CompilerParams(
    needs_layout_passes=False, use_tc_tiling_on_sc=False)


def _sum_body(table, src, dst, zinit, out_s, out_d, sidx0, dvec0, msg0,
              sidx1, dvec1, msg1, deg, shared, sem0, sem1):
    c = lax.axis_index("c")
    s = lax.axis_index("s")

    @pl.when(s == 0)
    def _():
        pltpu.sync_copy(zinit, shared)

    zero16 = jnp.zeros((16,), jnp.float32)

    def zdeg(i, _):
        deg[pl.ds(i * 16, 16)] = zero16
        return 0

    lax.fori_loop(0, _N // 16, zdeg, 0)
    plsc.subcore_barrier()

    ebase = s * _ETS
    ones16 = jnp.ones((16,), jnp.float32)
    chunks = ((0, 128), (128, 128), (256, 128), (384, 16))
    bufs = ((sidx0, dvec0, msg0, sem0), (sidx1, dvec1, msg1, sem1))

    def stage(b, sidx, dvec, msg, sem):
        base = ebase + b * _BS
        pltpu.sync_copy(src.at[pl.ds(base, _BS)], sidx)
        pltpu.sync_copy(dst.at[pl.ds(base, _BS)], dvec)

        def adj(i, _):
            for j in range(5):
                v = sidx[pl.ds((i * 5 + j) * 16, 16)]
                sidx[pl.ds((i * 5 + j) * 16, 16)] = v * 2 + c
            return 0

        lax.fori_loop(0, 5, adj, 0)
        for k, sz in chunks:
            pltpu.async_copy(table.at[sidx.at[pl.ds(k, sz)]],
                             msg.at[pl.ds(k, sz)], sem)

    def process(sidx, dvec, msg, sem):
        for k, sz in chunks:
            pltpu.make_async_copy(table.at[sidx.at[pl.ds(k, sz)]],
                                  msg.at[pl.ds(k, sz)], sem).wait()
        for k, sz in chunks:
            pltpu.sync_copy(msg.at[pl.ds(k, sz)],
                            shared.at[dvec.at[pl.ds(k, sz)]], add=True)

        @pl.when(c == 0)
        def _():
            def grpd(g, _):
                d = dvec[pl.ds(g * 16, 16)]
                plsc.addupdate_scatter(deg, [d], ones16)
                return 0

            lax.fori_loop(0, _BS // 16, grpd, 0)

    npair = _ETS // _BS // 2
    stage(0, *bufs[0])

    def pair(i, _):
        stage(i * 2 + 1, *bufs[1])
        process(*bufs[0])

        @pl.when(i < npair - 1)
        def _():
            stage(i * 2 + 2, *bufs[0])

        process(*bufs[1])
        return 0

    lax.fori_loop(0, npair, pair, 0)
    plsc.subcore_barrier()

    @pl.when(s == 0)
    def _():
        pltpu.sync_copy(shared, out_s.at[pl.ds(0, _N), pl.ds(c * 64, 64)])

    @pl.when(c == 0)
    def _():
        pltpu.sync_copy(deg, out_d.at[s])


def _min_body(table, src, dst, iinit, out_m, acc, tmp,
              sidx0, dvec0, msg0, sidx1, dvec1, msg1, sidx2, dvec2, msg2,
              semi0, semi1, semi2, semg0, semg1, semg2):
    c = lax.axis_index("c")
    s = lax.axis_index("s")
    fc = s
    eh = c
    pltpu.sync_copy(iinit, acc)
    iota = lax.iota(jnp.int32, 16)
    ebase = eh * _EH
    bufs = ((sidx0, dvec0, msg0, semi0, semg0),
            (sidx1, dvec1, msg1, semi1, semg1),
            (sidx2, dvec2, msg2, semi2, semg2))

    def fire_idx(b, slot):
        sidx, dvec, _, semi, _ = bufs[slot]
        base = ebase + b * _BF
        pltpu.async_copy(src.at[pl.ds(base, _BF)], sidx, semi)
        pltpu.async_copy(dst.at[pl.ds(base, _BF)], dvec, semi)

    def phase_a(b, slot):
        sidx, dvec, msg, semi, semg = bufs[slot]
        base = ebase + b * _BF
        pltpu.make_async_copy(src.at[pl.ds(base, _BF)], sidx, semi).wait()
        pltpu.make_async_copy(dst.at[pl.ds(base, _BF)], dvec, semi).wait()

        def adj(i, _):
            for j in range(4):
                v = sidx[pl.ds((i * 4 + j) * 16, 16)]
                sidx[pl.ds((i * 4 + j) * 16, 16)] = v * 16 + fc
            return 0

        lax.fori_loop(0, _BF // 64, adj, 0)
        for k in range(_BF // 128):
            pltpu.async_copy(
                table.at[sidx.at[pl.ds(k * 128, 128)]],
                msg.at[pl.ds(k * 128, 128)], semg)

    def accum(dvec, msg):
        npg = _BF // 32
        iota16 = iota + 16

        def checkpair(pi):
            dA = dvec[pl.ds(pi * 32, 16)]
            dB = dvec[pl.ds(pi * 32 + 16, 16)]
            plsc.store_scatter(tmp, [dA], iota)
            plsc.store_scatter(tmp, [dB], iota16)
            rbA = plsc.load_gather(tmp, [dA])
            rbB = plsc.load_gather(tmp, [dB])
            return dA, dB, jnp.any((rbA != iota) | (rbB != iota16))

        def grp(pi, carry):
            dA, dB, dup = carry
            nxt = checkpair(jnp.minimum(pi + 1, npg - 1))
            eA = pi * 32 + iota
            eB = eA + 16

            @pl.when(jnp.logical_not(dup))
            def _():
                @plsc.parallel_loop(0, 8, step=1, unroll=8)
                def _(f):
                    fv = jnp.full((16,), f, jnp.int32)
                    colA = plsc.load_gather(msg, [eA, fv])
                    colB = plsc.load_gather(msg, [eB, fv])
                    curA = plsc.load_gather(acc, [dA, fv])
                    curB = plsc.load_gather(acc, [dB, fv])
                    plsc.store_scatter(acc, [dA, fv], jnp.minimum(curA, colA))
                    plsc.store_scatter(acc, [dB, fv], jnp.minimum(curB, colB))

            @pl.when(dup)
            def _():
                for d, e, ids in ((dA, eA, iota), (dB, eB, iota16)):
                    plsc.store_scatter(tmp, [d], ids)
                    rbg = plsc.load_gather(tmp, [d])
                    gdup = jnp.any(rbg != ids)

                    @pl.when(jnp.logical_not(gdup))
                    def _():
                        @plsc.parallel_loop(0, 8, step=1, unroll=8)
                        def _(f):
                            fv = jnp.full((16,), f, jnp.int32)
                            col = plsc.load_gather(msg, [e, fv])
                            cur = plsc.load_gather(acc, [d, fv])
                            plsc.store_scatter(acc, [d, fv],
                                               jnp.minimum(cur, col))

                    @pl.when(gdup)
                    def _():
                        for f in range(8):
                            fv = jnp.full((16,), f, jnp.int32)
                            col = plsc.load_gather(msg, [e, fv])

                            def body(a):
                                plsc.store_scatter(acc, [d, fv], col, mask=a)
                                rb2 = plsc.load_gather(acc, [d, fv])
                                return a & (col < rb2)

                            a0 = col < plsc.load_gather(acc, [d, fv])
                            lax.while_loop(lambda a: jnp.any(a), body, a0)

            return nxt

        lax.fori_loop(0, npg, grp, checkpair(0))

    def phase_b(b, slot):
        sidx, dvec, msg, _, semg = bufs[slot]
        base = ebase + b * _BF
        del base
        for k in range(_BF // 128):
            pltpu.make_async_copy(
                table.at[sidx.at[pl.ds(k * 128, 128)]],
                msg.at[pl.ds(k * 128, 128)], semg).wait()
        accum(dvec, msg)

    fire_idx(0, 0)
    fire_idx(1, 1)
    phase_a(0, 0)

    def triple(i, _):
        for j in range(3):
            b = i * 3 + j
            fire_idx(b + 2, (j + 2) % 3)
            phase_a(b + 1, (j + 1) % 3)
            phase_b(b, j)
        return 0

    lax.fori_loop(0, (_NB - 2) // 3, triple, 0)
    phase_a(_NB - 1, (_NB - 1) % 3)
    phase_b(_NB - 2, (_NB - 2) % 3)
    phase_b(_NB - 1, (_NB - 1) % 3)

    pltpu.sync_copy(acc, out_m.at[eh, pl.ds(0, _N), pl.ds(fc * 8, 8)])


_seg_sum = pl.kernel(
    _sum_body,
    out_type=(jax.ShapeDtypeStruct((_N, _HID), jnp.float32),
              jax.ShapeDtypeStruct((16, _N), jnp.float32)),
    mesh=_mesh,
    compiler_params=_sc_params,
    scratch_types=[
        pltpu.VMEM((_BS,), jnp.int32),
        pltpu.VMEM((_BS,), jnp.int32),
        pltpu.VMEM((_BS, 64), jnp.float32),
        pltpu.VMEM((_BS,), jnp.int32),
        pltpu.VMEM((_BS,), jnp.int32),
        pltpu.VMEM((_BS, 64), jnp.float32),
        pltpu.VMEM((_N,), jnp.float32),
        pltpu.VMEM_SHARED((_N, 64), jnp.float32),
        pltpu.SemaphoreType.DMA,
        pltpu.SemaphoreType.DMA,
    ],
)

_seg_min = pl.kernel(
    _min_body,
    out_type=jax.ShapeDtypeStruct((2, _N, _HID), jnp.float32),
    mesh=_mesh,
    compiler_params=_sc_params,
    scratch_types=[
        pltpu.VMEM((_N, 8), jnp.float32),
        pltpu.VMEM((_N,), jnp.int32),
        pltpu.VMEM((_BF,), jnp.int32),
        pltpu.VMEM((_BF,), jnp.int32),
        pltpu.VMEM((_BF, 8), jnp.float32),
        pltpu.VMEM((_BF,), jnp.int32),
        pltpu.VMEM((_BF,), jnp.int32),
        pltpu.VMEM((_BF, 8), jnp.float32),
        pltpu.VMEM((_BF,), jnp.int32),
        pltpu.VMEM((_BF,), jnp.int32),
        pltpu.VMEM((_BF, 8), jnp.float32),
        pltpu.SemaphoreType.DMA,
        pltpu.SemaphoreType.DMA,
        pltpu.SemaphoreType.DMA,
        pltpu.SemaphoreType.DMA,
        pltpu.SemaphoreType.DMA,
        pltpu.SemaphoreType.DMA,
    ],
)


def _mm_tanh_body(x_ref, w_ref, b_ref, o_ref):
    o_ref[...] = jnp.tanh(
        jnp.dot(x_ref[...], w_ref[...], preferred_element_type=jnp.float32)
        + b_ref[...])


def _mid_body(sp_ref, dg_ref, w_ref, b_ref, o_ref):
    s = sp_ref[...]
    deg = jnp.sum(dg_ref[...], axis=0)
    m = s / jnp.maximum(deg, 1.0)[:, None]
    o_ref[...] = jnp.tanh(
        jnp.dot(m, w_ref[...], preferred_element_type=jnp.float32)
        + b_ref[...])


def _cls_body(ap_ref, w1_ref, b1_ref, w2_ref, b2_ref, o_ref):
    agg = jnp.minimum(ap_ref[0], ap_ref[1])
    c1 = jnp.tanh(
        jnp.dot(agg, w1_ref[...], preferred_element_type=jnp.float32)
        + b1_ref[...])
    o_ref[...] = (
        jnp.dot(c1, w2_ref[...], preferred_element_type=jnp.float32)
        + b2_ref[...])


_mm_tanh = pl.pallas_call(
    _mm_tanh_body,
    out_shape=jax.ShapeDtypeStruct((_N, _HID), jnp.float32),
)

_mid = pl.pallas_call(
    _mid_body,
    out_shape=jax.ShapeDtypeStruct((_N, _HID), jnp.float32),
)

_cls = pl.pallas_call(
    _cls_body,
    out_shape=jax.ShapeDtypeStruct((_N, _OUT), jnp.float32),
)


def kernel(x, x_struct, x_e, edge_index, W1, b1, W2, b2, Wc1, bc1, Wc2, bc2):
    del x_struct, x_e
    src = edge_index[0]
    dst = edge_index[1]

    h1 = _mm_tanh(x, W1, b1.reshape(1, _HID))
    zinit = jnp.zeros((_N, 64), jnp.float32)
    s_part, d_part = _seg_sum(h1.reshape(_N * 2, 64), src, dst, zinit)
    h2 = _mid(s_part, d_part, W2, b2.reshape(1, _HID))
    iinit = jnp.full((_N, 8), jnp.inf, jnp.float32)
    a_part = _seg_min(h2.reshape(_N * 16, 8), src, dst, iinit)
    out = _cls(a_part, Wc1, bc1.reshape(1, _HID), Wc2, bc2.reshape(1, _OUT))
    return out


# speedup vs baseline: 1.2774x; 1.0004x over previous
import jax
import jax.numpy as jnp
from jax import lax
from jax.experimental import pallas as pl
from jax.experimental.pallas import tpu as pltpu
from jax.experimental.pallas import tpu_sc as plsc

_N = 10000
_E = 320000
_HID = 128
_OUT = 64

_EH = _E // 2
_ETS = _E // 16
_BS = 400
_BF = 1280
_NB = 125

_mesh = plsc.VectorSubcoreMesh(
    core_axis_name="c", subcore_axis_name="s", num_cores=2, num_subcores=16)

_sc_params = pltpu.CompilerParams(
    needs_layout_passes=False, use_tc_tiling_on_sc=False)


def _sum_body(table, src, dst, zinit, out_s, out_d, sidx0, dvec0, msg0,
              sidx1, dvec1, msg1, deg, shared, sem0, sem1):
    c = lax.axis_index("c")
    s = lax.axis_index("s")

    @pl.when(s == 0)
    def _():
        pltpu.sync_copy(zinit, shared)

    zero16 = jnp.zeros((16,), jnp.float32)

    def zdeg(i, _):
        deg[pl.ds(i * 16, 16)] = zero16
        return 0

    lax.fori_loop(0, _N // 16, zdeg, 0)
    plsc.subcore_barrier()

    ebase = s * _ETS
    ones16 = jnp.ones((16,), jnp.float32)
    chunks = ((0, 128), (128, 128), (256, 128), (384, 16))
    bufs = ((sidx0, dvec0, msg0, sem0), (sidx1, dvec1, msg1, sem1))

    def stage(b, sidx, dvec, msg, sem):
        base = ebase + b * _BS
        pltpu.sync_copy(src.at[pl.ds(base, _BS)], sidx)
        pltpu.sync_copy(dst.at[pl.ds(base, _BS)], dvec)

        def adj(i, _):
            for j in range(5):
                v = sidx[pl.ds((i * 5 + j) * 16, 16)]
                sidx[pl.ds((i * 5 + j) * 16, 16)] = v * 2 + c
            return 0

        lax.fori_loop(0, 5, adj, 0)
        for k, sz in chunks:
            pltpu.async_copy(table.at[sidx.at[pl.ds(k, sz)]],
                             msg.at[pl.ds(k, sz)], sem)

    def process(sidx, dvec, msg, sem):
        for k, sz in chunks:
            pltpu.make_async_copy(table.at[sidx.at[pl.ds(k, sz)]],
                                  msg.at[pl.ds(k, sz)], sem).wait()
        for k, sz in chunks:
            pltpu.sync_copy(msg.at[pl.ds(k, sz)],
                            shared.at[dvec.at[pl.ds(k, sz)]], add=True)

        @pl.when(c == 0)
        def _():
            def grpd(g, _):
                d = dvec[pl.ds(g * 16, 16)]
                plsc.addupdate_scatter(deg, [d], ones16)
                return 0

            lax.fori_loop(0, _BS // 16, grpd, 0)

    npair = _ETS // _BS // 2
    stage(0, *bufs[0])

    def pair(i, _):
        stage(i * 2 + 1, *bufs[1])
        process(*bufs[0])

        @pl.when(i < npair - 1)
        def _():
            stage(i * 2 + 2, *bufs[0])

        process(*bufs[1])
        return 0

    lax.fori_loop(0, npair, pair, 0)
    plsc.subcore_barrier()

    @pl.when(s == 0)
    def _():
        pltpu.sync_copy(shared, out_s.at[pl.ds(0, _N), pl.ds(c * 64, 64)])

    @pl.when(c == 0)
    def _():
        pltpu.sync_copy(deg, out_d.at[s])


def _min_body(table, src, dst, iinit, out_m, acc, tmp,
              sidx0, dvec0, msg0, sidx1, dvec1, msg1, sidx2, dvec2, msg2,
              semi0, semi1, semi2, semg0, semg1, semg2):
    c = lax.axis_index("c")
    s = lax.axis_index("s")
    fc = s
    eh = c
    pltpu.sync_copy(iinit, acc)
    iota = lax.iota(jnp.int32, 16)
    ebase = eh * _EH
    bufs = ((sidx0, dvec0, msg0, semi0, semg0),
            (sidx1, dvec1, msg1, semi1, semg1),
            (sidx2, dvec2, msg2, semi2, semg2))

    def fire_idx(b, slot):
        sidx, dvec, _, semi, _ = bufs[slot]
        base = ebase + b * _BF
        pltpu.async_copy(src.at[pl.ds(base, _BF)], sidx, semi)
        pltpu.async_copy(dst.at[pl.ds(base, _BF)], dvec, semi)

    def phase_a(b, slot):
        sidx, dvec, msg, semi, semg = bufs[slot]
        base = ebase + b * _BF
        pltpu.make_async_copy(src.at[pl.ds(base, _BF)], sidx, semi).wait()
        pltpu.make_async_copy(dst.at[pl.ds(base, _BF)], dvec, semi).wait()

        def adj(i, _):
            for j in range(4):
                v = sidx[pl.ds((i * 4 + j) * 16, 16)]
                sidx[pl.ds((i * 4 + j) * 16, 16)] = v * 16 + fc
            return 0

        lax.fori_loop(0, _BF // 64, adj, 0)
        for k in range(_BF // 128):
            pltpu.async_copy(
                table.at[sidx.at[pl.ds(k * 128, 128)]],
                msg.at[pl.ds(k * 128, 128)], semg)

    def accum(dvec, msg):
        npg = _BF // 32
        iota16 = iota + 16

        def checkpair(pi):
            dA = dvec[pl.ds(pi * 32, 16)]
            dB = dvec[pl.ds(pi * 32 + 16, 16)]
            plsc.store_scatter(tmp, [dA], iota)
            plsc.store_scatter(tmp, [dB], iota16)
            rbA = plsc.load_gather(tmp, [dA])
            rbB = plsc.load_gather(tmp, [dB])
            return dA, dB, jnp.any((rbA != iota) | (rbB != iota16))

        def grp(pi, carry):
            dA, dB, dup = carry
            nxt = checkpair(jnp.minimum(pi + 1, npg - 1))
            eA = pi * 32 + iota
            eB = eA + 16

            @pl.when(jnp.logical_not(dup))
            def _():
                @plsc.parallel_loop(0, 8, step=1, unroll=8)
                def _(f):
                    fv = jnp.full((16,), f, jnp.int32)
                    colA = plsc.load_gather(msg, [eA, fv])
                    colB = plsc.load_gather(msg, [eB, fv])
                    curA = plsc.load_gather(acc, [dA, fv])
                    curB = plsc.load_gather(acc, [dB, fv])
                    plsc.store_scatter(acc, [dA, fv], jnp.minimum(curA, colA))
                    plsc.store_scatter(acc, [dB, fv], jnp.minimum(curB, colB))

            @pl.when(dup)
            def _():
                for d, e, ids in ((dA, eA, iota), (dB, eB, iota16)):
                    plsc.store_scatter(tmp, [d], ids)
                    rbg = plsc.load_gather(tmp, [d])
                    gdup = jnp.any(rbg != ids)

                    @pl.when(jnp.logical_not(gdup))
                    def _():
                        @plsc.parallel_loop(0, 8, step=1, unroll=8)
                        def _(f):
                            fv = jnp.full((16,), f, jnp.int32)
                            col = plsc.load_gather(msg, [e, fv])
                            cur = plsc.load_gather(acc, [d, fv])
                            plsc.store_scatter(acc, [d, fv],
                                               jnp.minimum(cur, col))

                    @pl.when(gdup)
                    def _():
                        for f in range(8):
                            fv = jnp.full((16,), f, jnp.int32)
                            col = plsc.load_gather(msg, [e, fv])

                            def body(a):
                                plsc.store_scatter(acc, [d, fv], col, mask=a)
                                rb2 = plsc.load_gather(acc, [d, fv])
                                return a & (col < rb2)

                            a0 = col < plsc.load_gather(acc, [d, fv])
                            lax.while_loop(lambda a: jnp.any(a), body, a0)

            return nxt

        lax.fori_loop(0, npg, grp, checkpair(0))

    def phase_b(b, slot):
        sidx, dvec, msg, _, semg = bufs[slot]
        for k in range(_BF // 128):
            pltpu.make_async_copy(
                table.at[sidx.at[pl.ds(k * 128, 128)]],
                msg.at[pl.ds(k * 128, 128)], semg).wait()
        accum(dvec, msg)

    fire_idx(0, 0)
    fire_idx(1, 1)
    phase_a(0, 0)

    def triple(i, _):
        for j in range(3):
            b = i * 3 + j
            fire_idx(b + 2, (j + 2) % 3)
            phase_a(b + 1, (j + 1) % 3)
            phase_b(b, j)
        return 0

    lax.fori_loop(0, (_NB - 2) // 3, triple, 0)
    phase_a(_NB - 1, (_NB - 1) % 3)
    phase_b(_NB - 2, (_NB - 2) % 3)
    phase_b(_NB - 1, (_NB - 1) % 3)

    pltpu.sync_copy(acc, out_m.at[eh, pl.ds(0, _N), pl.ds(fc * 8, 8)])


_seg_sum = pl.kernel(
    _sum_body,
    out_type=(jax.ShapeDtypeStruct((_N, _HID), jnp.float32),
              jax.ShapeDtypeStruct((16, _N), jnp.float32)),
    mesh=_mesh,
    compiler_params=_sc_params,
    scratch_types=[
        pltpu.VMEM((_BS,), jnp.int32),
        pltpu.VMEM((_BS,), jnp.int32),
        pltpu.VMEM((_BS, 64), jnp.float32),
        pltpu.VMEM((_BS,), jnp.int32),
        pltpu.VMEM((_BS,), jnp.int32),
        pltpu.VMEM((_BS, 64), jnp.float32),
        pltpu.VMEM((_N,), jnp.float32),
        pltpu.VMEM_SHARED((_N, 64), jnp.float32),
        pltpu.SemaphoreType.DMA,
        pltpu.SemaphoreType.DMA,
    ],
)

_seg_min = pl.kernel(
    _min_body,
    out_type=jax.ShapeDtypeStruct((2, _N, _HID), jnp.float32),
    mesh=_mesh,
    compiler_params=_sc_params,
    scratch_types=[
        pltpu.VMEM((_N, 8), jnp.float32),
        pltpu.VMEM((_N,), jnp.int32),
        pltpu.VMEM((_BF,), jnp.int32),
        pltpu.VMEM((_BF,), jnp.int32),
        pltpu.VMEM((_BF, 8), jnp.float32),
        pltpu.VMEM((_BF,), jnp.int32),
        pltpu.VMEM((_BF,), jnp.int32),
        pltpu.VMEM((_BF, 8), jnp.float32),
        pltpu.VMEM((_BF,), jnp.int32),
        pltpu.VMEM((_BF,), jnp.int32),
        pltpu.VMEM((_BF, 8), jnp.float32),
        pltpu.SemaphoreType.DMA,
        pltpu.SemaphoreType.DMA,
        pltpu.SemaphoreType.DMA,
        pltpu.SemaphoreType.DMA,
        pltpu.SemaphoreType.DMA,
        pltpu.SemaphoreType.DMA,
    ],
)


def _mm_tanh_body(x_ref, w_ref, b_ref, o_ref):
    o_ref[...] = jnp.tanh(
        jnp.dot(x_ref[...], w_ref[...], preferred_element_type=jnp.float32)
        + b_ref[...])


def _mid_body(sp_ref, dg_ref, w_ref, b_ref, o_ref):
    s = sp_ref[...]
    deg = jnp.sum(dg_ref[...], axis=0)
    m = s / jnp.maximum(deg, 1.0)[:, None]
    o_ref[...] = jnp.tanh(
        jnp.dot(m, w_ref[...], preferred_element_type=jnp.float32)
        + b_ref[...])


def _cls_body(ap_ref, w1_ref, b1_ref, w2_ref, b2_ref, o_ref):
    agg = jnp.minimum(ap_ref[0], ap_ref[1])
    c1 = jnp.tanh(
        jnp.dot(agg, w1_ref[...], preferred_element_type=jnp.float32)
        + b1_ref[...])
    o_ref[...] = (
        jnp.dot(c1, w2_ref[...], preferred_element_type=jnp.float32)
        + b2_ref[...])


_mm_tanh = pl.pallas_call(
    _mm_tanh_body,
    out_shape=jax.ShapeDtypeStruct((_N, _HID), jnp.float32),
)

_mid = pl.pallas_call(
    _mid_body,
    out_shape=jax.ShapeDtypeStruct((_N, _HID), jnp.float32),
)

_cls = pl.pallas_call(
    _cls_body,
    out_shape=jax.ShapeDtypeStruct((_N, _OUT), jnp.float32),
)


def kernel(x, x_struct, x_e, edge_index, W1, b1, W2, b2, Wc1, bc1, Wc2, bc2):
    del x_struct, x_e
    src = edge_index[0]
    dst = edge_index[1]

    h1 = _mm_tanh(x, W1, b1.reshape(1, _HID))
    zinit = jnp.zeros((_N, 64), jnp.float32)
    s_part, d_part = _seg_sum(h1.reshape(_N * 2, 64), src, dst, zinit)
    h2 = _mid(s_part, d_part, W2, b2.reshape(1, _HID))
    iinit = jnp.full((_N, 8), jnp.inf, jnp.float32)
    a_part = _seg_min(h2.reshape(_N * 16, 8), src, dst, iinit)
    out = _cls(a_part, Wc1, bc1.reshape(1, _HID), Wc2, bc2.reshape(1, _OUT))
    return out
